# baseline (speedup 1.0000x reference)
import functools

import jax
import jax.numpy as jnp
from jax import lax
from jax.experimental import pallas as pl
from jax.experimental.pallas import tpu as pltpu
from jax.experimental.pallas import tpu_sc as plsc

MAIN_DIM = 16
SUB_DIM = 48
OUT_DIM = MAIN_DIM + SUB_DIM

NUM_CORES = 2
NUM_SUBCORES = 16
NUM_WORKERS = NUM_CORES * NUM_SUBCORES

GH = 8
GB = 128
QH = 2
NQ = GH // QH
LG = GB // 16


def _encoder(n_hist, n_batch, n_main, n_sub):
    ht_n = n_hist // GH
    bt_n = n_batch // GB
    n_groups = ht_n * bt_n
    per_w = n_groups // NUM_WORKERS
    assert n_groups % NUM_WORKERS == 0
    mesh = plsc.VectorSubcoreMesh(core_axis_name="c", subcore_axis_name="s")

    @functools.partial(
        pl.kernel,
        mesh=mesh,
        compiler_params=pltpu.CompilerParams(use_tc_tiling_on_sc=False,
                                             needs_layout_passes=False),
        out_type=jax.ShapeDtypeStruct((n_hist, OUT_DIM // 8, bt_n, 8, GB),
                                      jnp.float32),
        scratch_types=[
            pltpu.VMEM((GH, GB), jnp.int32),
            pltpu.VMEM((GH, GB), jnp.int32),
            pltpu.VMEM((n_main, MAIN_DIM), jnp.float32),
            pltpu.VMEM((2 * QH * GB, SUB_DIM), jnp.float32),
            pltpu.VMEM((2, OUT_DIM // 8, 8, GB), jnp.float32),
            pltpu.SemaphoreType.DMA,
            pltpu.SemaphoreType.DMA,
            pltpu.SemaphoreType.DMA,
            pltpu.SemaphoreType.DMA,
        ],
    )
    def enc(mid4_hbm, sid4_hbm, mtab_hbm, stab_hbm, out_hbm,
            idx_m, idx_s, mainv, sbuf, obuf,
            sem_g0, sem_g1, sem_o0, sem_o1):
        wid = lax.axis_index("s") * NUM_CORES + lax.axis_index("c")
        sem_g = (sem_g0, sem_g1)
        sem_o = (sem_o0, sem_o1)
        lanes = lax.iota(jnp.int32, 16)

        pltpu.sync_copy(mtab_hbm, mainv)

        for ob in range(2):
            pltpu.async_copy(out_hbm.at[0, :, 0], obuf.at[ob], sem_o[ob])

        def gather_q(q):
            half = q % 2
            for j in range(QH):
                pltpu.async_copy(
                    stab_hbm.at[idx_s.at[q * QH + j]],
                    sbuf.at[pl.ds((half * QH + j) * GB, GB)],
                    sem_g[half])

        def drain_gather_q(q):
            half = q % 2
            for j in range(QH):
                pltpu.make_async_copy(
                    stab_hbm.at[idx_s.at[q * QH + j]],
                    sbuf.at[pl.ds((half * QH + j) * GB, GB)],
                    sem_g[half]).wait()

        def body(k, _):
            g = wid * per_w + k
            ht = g // bt_n
            bt = g % bt_n
            h0 = ht * GH

            pltpu.sync_copy(mid4_hbm.at[ht, bt], idx_m)
            pltpu.sync_copy(sid4_hbm.at[ht, bt], idx_s)
            gather_q(0)

            for q in range(NQ):
                if q + 1 < NQ:
                    gather_q(q + 1)
                drain_gather_q(q)
                half = q % 2
                for i in range(QH):
                    hr = q * QH + i
                    ob = i
                    pltpu.make_async_copy(
                        out_hbm.at[h0 + hr, :, bt],
                        obuf.at[ob], sem_o[ob]).wait()

                    def fill_lg(lg, _, *, hr=hr, half=half, i=i, ob=ob):
                        off = lg * 16
                        idv = idx_m[hr, pl.ds(off, 16)]
                        rvec = (half * QH + i) * GB + off + lanes
                        for f in range(MAIN_DIM):
                            val = plsc.load_gather(
                                mainv,
                                [idv, jnp.full((16,), f, jnp.int32)])
                            obuf[ob, f // 8, f % 8, pl.ds(off, 16)] = val
                        for f in range(SUB_DIM):
                            val = plsc.load_gather(
                                sbuf,
                                [rvec, jnp.full((16,), f, jnp.int32)])
                            fo = MAIN_DIM + f
                            obuf[ob, fo // 8, fo % 8, pl.ds(off, 16)] = val
                        return 0

                    lax.fori_loop(0, LG, fill_lg, 0)
                    pltpu.async_copy(
                        obuf.at[ob], out_hbm.at[h0 + hr, :, bt], sem_o[ob])
            return 0

        lax.fori_loop(0, per_w, body, 0)

        for ob in range(2):
            pltpu.make_async_copy(out_hbm.at[0, :, 0], obuf.at[ob],
                                  sem_o[ob]).wait()

    return enc


def kernel(main_category_id, sub_category_id, main_table, sub_table):
    b, h = main_category_id.shape
    ht_n, bt_n = h // GH, b // GB

    def as_blocks(ids):
        return (ids.T.astype(jnp.int32)
                .reshape(ht_n, GH, bt_n, GB)
                .transpose(0, 2, 1, 3))

    out5 = _encoder(h, b, main_table.shape[0], sub_table.shape[0])(
        as_blocks(main_category_id), as_blocks(sub_category_id),
        main_table, sub_table)
    out = (out5.transpose(0, 1, 3, 2, 4)
           .reshape(h, OUT_DIM, b)
           .transpose(2, 0, 1))
    return out


# speedup vs baseline: 1.4589x; 1.4589x over previous
import functools

import jax
import jax.numpy as jnp
from jax import lax
from jax.experimental import pallas as pl
from jax.experimental.pallas import tpu as pltpu
from jax.experimental.pallas import tpu_sc as plsc

MAIN_DIM = 16
SUB_DIM = 48
OUT_DIM = MAIN_DIM + SUB_DIM

NUM_CORES = 2
NUM_SUBCORES = 16
NUM_WORKERS = NUM_CORES * NUM_SUBCORES

GH = 8
GB = 128
QH = 2
NQ = GH // QH
LG = GB // 16


def _encoder(n_hist, n_batch, n_main, n_sub):
    ht_n = n_hist // GH
    bt_n = n_batch // GB
    n_groups = ht_n * bt_n
    per_w = n_groups // NUM_WORKERS
    assert n_groups % NUM_WORKERS == 0
    mesh = plsc.VectorSubcoreMesh(core_axis_name="c", subcore_axis_name="s")

    @functools.partial(
        pl.kernel,
        mesh=mesh,
        compiler_params=pltpu.CompilerParams(use_tc_tiling_on_sc=False,
                                             needs_layout_passes=False),
        out_type=jax.ShapeDtypeStruct((n_hist, OUT_DIM // 8, bt_n, 8, GB),
                                      jnp.float32),
        scratch_types=[
            pltpu.VMEM((GH, GB), jnp.int32),
            pltpu.VMEM((GH, GB), jnp.int32),
            pltpu.VMEM((n_main, MAIN_DIM), jnp.float32),
            pltpu.VMEM((n_main, MAIN_DIM + 1), jnp.float32),
            pltpu.VMEM((2 * QH * GB, SUB_DIM), jnp.float32),
            pltpu.VMEM((2, OUT_DIM // 8, 8, GB + 1), jnp.float32),
            pltpu.SemaphoreType.DMA,
            pltpu.SemaphoreType.DMA,
            pltpu.SemaphoreType.DMA,
            pltpu.SemaphoreType.DMA,
        ],
    )
    def enc(mid4_hbm, sid4_hbm, mtab_hbm, stab_hbm, out_hbm,
            idx_m, idx_s, mstage, mainv, sbuf, obuf,
            sem_g0, sem_g1, sem_o0, sem_o1):
        wid = lax.axis_index("s") * NUM_CORES + lax.axis_index("c")
        sem_g = (sem_g0, sem_g1)
        sem_o = (sem_o0, sem_o1)
        lanes = lax.iota(jnp.int32, 16)

        pltpu.sync_copy(mtab_hbm, mstage)

        def pad_main(r, _):
            mainv[r, pl.ds(0, MAIN_DIM)] = mstage[r, pl.ds(0, MAIN_DIM)]
            return 0

        lax.fori_loop(0, n_main, pad_main, 0)

        for ob in range(2):
            pltpu.async_copy(out_hbm.at[0, :, 0],
                             obuf.at[ob, :, :, pl.ds(0, GB)], sem_o[ob])

        def gather_q(q):
            half = q % 2
            for j in range(QH):
                pltpu.async_copy(
                    stab_hbm.at[idx_s.at[q * QH + j]],
                    sbuf.at[pl.ds((half * QH + j) * GB, GB)],
                    sem_g[half])

        def drain_gather_q(q):
            half = q % 2
            for j in range(QH):
                pltpu.make_async_copy(
                    stab_hbm.at[idx_s.at[q * QH + j]],
                    sbuf.at[pl.ds((half * QH + j) * GB, GB)],
                    sem_g[half]).wait()

        def body(k, _):
            g = wid * per_w + k
            ht = g // bt_n
            bt = g % bt_n
            h0 = ht * GH

            pltpu.sync_copy(mid4_hbm.at[ht, bt], idx_m)
            pltpu.sync_copy(sid4_hbm.at[ht, bt], idx_s)
            gather_q(0)

            for q in range(NQ):
                if q + 1 < NQ:
                    gather_q(q + 1)
                drain_gather_q(q)
                half = q % 2
                for i in range(QH):
                    hr = q * QH + i
                    ob = i
                    pltpu.make_async_copy(
                        out_hbm.at[h0 + hr, :, bt],
                        obuf.at[ob, :, :, pl.ds(0, GB)], sem_o[ob]).wait()

                    def fill_main(lg, _, *, hr=hr, ob=ob):
                        off = lg * 16
                        idv = idx_m[hr, pl.ds(off, 16)]
                        for f in range(MAIN_DIM):
                            val = plsc.load_gather(
                                mainv,
                                [idv, jnp.full((16,), f, jnp.int32)])
                            obuf[ob, f // 8, f % 8, pl.ds(off, 16)] = val
                        return 0

                    def fill_sub(r, _, *, half=half, i=i, ob=ob):
                        row = (half * QH + i) * GB + r
                        bvec = jnp.broadcast_to(r, (16,))
                        obv = jnp.full((16,), ob, jnp.int32)
                        for fc in range(SUB_DIM // 16):
                            v = sbuf[row, pl.ds(fc * 16, 16)]
                            fo = MAIN_DIM + fc * 16 + lanes
                            plsc.store_scatter(
                                obuf, [obv, fo // 8, fo % 8, bvec], v)
                        return 0

                    lax.fori_loop(0, LG, fill_main, 0)
                    lax.fori_loop(0, GB, fill_sub, 0)
                    pltpu.async_copy(
                        obuf.at[ob, :, :, pl.ds(0, GB)],
                        out_hbm.at[h0 + hr, :, bt], sem_o[ob])
            return 0

        lax.fori_loop(0, per_w, body, 0)

        for ob in range(2):
            pltpu.make_async_copy(out_hbm.at[0, :, 0],
                                  obuf.at[ob, :, :, pl.ds(0, GB)],
                                  sem_o[ob]).wait()

    return enc


def kernel(main_category_id, sub_category_id, main_table, sub_table):
    b, h = main_category_id.shape
    ht_n, bt_n = h // GH, b // GB

    def as_blocks(ids):
        return (ids.T.astype(jnp.int32)
                .reshape(ht_n, GH, bt_n, GB)
                .transpose(0, 2, 1, 3))

    out5 = _encoder(h, b, main_table.shape[0], sub_table.shape[0])(
        as_blocks(main_category_id), as_blocks(sub_category_id),
        main_table, sub_table)
    out = (out5.transpose(0, 1, 3, 2, 4)
           .reshape(h, OUT_DIM, b)
           .transpose(2, 0, 1))
    return out


# speedup vs baseline: 1.5068x; 1.0329x over previous
import functools

import jax
import jax.numpy as jnp
from jax import lax
from jax.experimental import pallas as pl
from jax.experimental.pallas import tpu as pltpu
from jax.experimental.pallas import tpu_sc as plsc

MAIN_DIM = 16
SUB_DIM = 48
OUT_DIM = MAIN_DIM + SUB_DIM

NUM_CORES = 2
NUM_SUBCORES = 16
NUM_WORKERS = NUM_CORES * NUM_SUBCORES

GH = 8
GB = 128
QH = 2
NQ = GH // QH
LG = GB // 16


def _encoder(n_hist, n_batch, n_main, n_sub):
    ht_n = n_hist // GH
    bt_n = n_batch // GB
    n_groups = ht_n * bt_n
    per_w = n_groups // NUM_WORKERS
    assert n_groups % NUM_WORKERS == 0
    mesh = plsc.VectorSubcoreMesh(core_axis_name="c", subcore_axis_name="s")

    @functools.partial(
        pl.kernel,
        mesh=mesh,
        compiler_params=pltpu.CompilerParams(use_tc_tiling_on_sc=False,
                                             needs_layout_passes=False),
        out_type=jax.ShapeDtypeStruct((n_hist, OUT_DIM // 8, bt_n, 8, GB),
                                      jnp.float32),
        scratch_types=[
            pltpu.VMEM((GH, GB), jnp.int32),
            pltpu.VMEM((GH, GB), jnp.int32),
            pltpu.VMEM((n_main, MAIN_DIM), jnp.float32),
            pltpu.VMEM((n_main, MAIN_DIM + 1), jnp.float32),
            pltpu.VMEM((2 * QH * GB, SUB_DIM), jnp.float32),
            pltpu.VMEM((2, OUT_DIM // 8, 8, GB + 1), jnp.float32),
            pltpu.SemaphoreType.DMA,
            pltpu.SemaphoreType.DMA,
            pltpu.SemaphoreType.DMA,
            pltpu.SemaphoreType.DMA,
        ],
    )
    def enc(mid4_hbm, sid4_hbm, mtab_hbm, stab_hbm, out_hbm,
            idx_m, idx_s, mstage, mainv, sbuf, obuf,
            sem_g0, sem_g1, sem_o0, sem_o1):
        wid = lax.axis_index("s") * NUM_CORES + lax.axis_index("c")
        sem_g = (sem_g0, sem_g1)
        sem_o = (sem_o0, sem_o1)
        lanes = lax.iota(jnp.int32, 16)

        pltpu.sync_copy(mtab_hbm, mstage)

        def pad_main(r, _):
            mainv[r, pl.ds(0, MAIN_DIM)] = mstage[r, pl.ds(0, MAIN_DIM)]
            return 0

        lax.fori_loop(0, n_main, pad_main, 0)

        for ob in range(2):
            pltpu.async_copy(out_hbm.at[0, :, 0],
                             obuf.at[ob, :, :, pl.ds(0, GB)], sem_o[ob])

        def gather_q(q):
            half = q % 2
            for j in range(QH):
                pltpu.async_copy(
                    stab_hbm.at[idx_s.at[q * QH + j]],
                    sbuf.at[pl.ds((half * QH + j) * GB, GB)],
                    sem_g[half])

        def drain_gather_q(q):
            half = q % 2
            for j in range(QH):
                pltpu.make_async_copy(
                    stab_hbm.at[idx_s.at[q * QH + j]],
                    sbuf.at[pl.ds((half * QH + j) * GB, GB)],
                    sem_g[half]).wait()

        def body(k, _):
            g = wid * per_w + k
            ht = g // bt_n
            bt = g % bt_n
            h0 = ht * GH

            pltpu.sync_copy(mid4_hbm.at[ht, bt], idx_m)
            pltpu.sync_copy(sid4_hbm.at[ht, bt], idx_s)
            gather_q(0)

            for q in range(NQ):
                if q + 1 < NQ:
                    gather_q(q + 1)
                drain_gather_q(q)
                half = q % 2
                for i in range(QH):
                    hr = q * QH + i
                    ob = i
                    pltpu.make_async_copy(
                        out_hbm.at[h0 + hr, :, bt],
                        obuf.at[ob, :, :, pl.ds(0, GB)], sem_o[ob]).wait()

                    def fill_main(lg, _, *, hr=hr, ob=ob):
                        off = lg * 16
                        idv = idx_m[hr, pl.ds(off, 16)]
                        for f in range(MAIN_DIM):
                            val = plsc.load_gather(
                                mainv,
                                [idv, jnp.full((16,), f, jnp.int32)])
                            obuf[ob, f // 8, f % 8, pl.ds(off, 16)] = val
                        return 0

                    def fill_sub(r4, _, *, half=half, i=i, ob=ob):
                        obv = jnp.full((16,), ob, jnp.int32)
                        for u in range(4):
                            r = r4 * 4 + u
                            row = (half * QH + i) * GB + r
                            bvec = jnp.broadcast_to(r, (16,))
                            for fc in range(SUB_DIM // 16):
                                v = sbuf[row, pl.ds(fc * 16, 16)]
                                fo = MAIN_DIM + fc * 16 + lanes
                                plsc.store_scatter(
                                    obuf, [obv, fo // 8, fo % 8, bvec], v)
                        return 0

                    lax.fori_loop(0, LG, fill_main, 0)
                    lax.fori_loop(0, GB // 4, fill_sub, 0)
                    pltpu.async_copy(
                        obuf.at[ob, :, :, pl.ds(0, GB)],
                        out_hbm.at[h0 + hr, :, bt], sem_o[ob])
            return 0

        lax.fori_loop(0, per_w, body, 0)

        for ob in range(2):
            pltpu.make_async_copy(out_hbm.at[0, :, 0],
                                  obuf.at[ob, :, :, pl.ds(0, GB)],
                                  sem_o[ob]).wait()

    return enc


def kernel(main_category_id, sub_category_id, main_table, sub_table):
    b, h = main_category_id.shape
    ht_n, bt_n = h // GH, b // GB

    def as_blocks(ids):
        return (ids.T.astype(jnp.int32)
                .reshape(ht_n, GH, bt_n, GB)
                .transpose(0, 2, 1, 3))

    out5 = _encoder(h, b, main_table.shape[0], sub_table.shape[0])(
        as_blocks(main_category_id), as_blocks(sub_category_id),
        main_table, sub_table)
    out = (out5.transpose(0, 1, 3, 2, 4)
           .reshape(h, OUT_DIM, b)
           .transpose(2, 0, 1))
    return out
